# Initial kernel scaffold; baseline (speedup 1.0000x reference)
import jax
import jax.numpy as jnp
from jax.experimental import pallas as pl


def kernel(node_feats, edge_attr, W_self0, W_neigh0, b0, g0, beta0, W_self1, W_neigh1, b1, g1, beta1, We1, be1, We2, be2, Wd1, bd1, Wd2, bd2, edge_index, src_nodes, dst_nodes):
    raise NotImplementedError("write your pallas kernel here")


# speedup vs baseline: 2.5278x; 2.5278x over previous
import functools

import jax
import jax.numpy as jnp
from jax import lax
from jax.experimental import pallas as pl
from jax.experimental.pallas import tpu as pltpu
from jax.experimental.pallas import tpu_sc as plsc

N_NODES = 50000
N_EDGES = 800000
NODE_DIM = 12
HIDDEN = 64
EMBED = 32
EDGE_REPR = 74

NC = 2
NS = 16
NW = NC * NS

NP = 50176
RPT = NP // NS
N_DUMMY = 176

E_PAD = 802816
EPW_SEG = E_PAD // NW
CHUNK = 128
NCHUNK_SEG = EPW_SEG // CHUNK

EPW_G = N_EDGES // NW
NFULL_G = EPW_G // CHUNK
TAIL_G = EPW_G - NFULL_G * CHUNK

_mesh = plsc.VectorSubcoreMesh(core_axis_name="c", subcore_axis_name="s")
_sc_params = pltpu.CompilerParams(use_tc_tiling_on_sc=False)


def _make_segsum(n_tables):
    @functools.partial(
        pl.kernel,
        out_type=jax.ShapeDtypeStruct((n_tables, NC, NP, 16), jnp.float32),
        mesh=_mesh,
        compiler_params=_sc_params,
        scratch_types=[
            pltpu.VMEM_SHARED((NP, 16), jnp.float32),
            pltpu.VMEM((RPT, 16), jnp.float32),
            pltpu.VMEM((CHUNK,), jnp.int32),
            pltpu.VMEM((CHUNK,), jnp.int32),
            pltpu.VMEM((CHUNK, 16), jnp.float32),
            pltpu.SemaphoreType.DMA,
        ],
    )
    def seg_kernel(*refs):
        tabs = refs[:n_tables]
        s_hbm, d_hbm, zeros_hbm, out_hbm = refs[n_tables:n_tables + 4]
        accum, stage, sidx, didx, rows, sem = refs[n_tables + 4:]
        cid = lax.axis_index("c")
        sid = lax.axis_index("s")
        base = (sid * NC + cid) * EPW_SEG
        for k in range(n_tables):
            pltpu.sync_copy(zeros_hbm, stage)
            pltpu.sync_copy(stage, accum.at[pl.ds(sid * RPT, RPT)])
            plsc.subcore_barrier()

            def chunk(i, carry):
                off = base + i * CHUNK
                pltpu.sync_copy(s_hbm.at[pl.ds(off, CHUNK)], sidx)
                pltpu.sync_copy(d_hbm.at[pl.ds(off, CHUNK)], didx)
                pltpu.async_copy(tabs[k].at[sidx], rows, sem).wait()
                pltpu.sync_copy(rows, accum.at[didx], add=True)
                return carry

            lax.fori_loop(0, NCHUNK_SEG, chunk, 0)
            plsc.subcore_barrier()
            pltpu.sync_copy(accum.at[pl.ds(sid * RPT, RPT)], stage)
            pltpu.sync_copy(stage, out_hbm.at[k, cid, pl.ds(sid * RPT, RPT)])
            plsc.subcore_barrier()

    return seg_kernel


@functools.partial(
    pl.kernel,
    out_type=jax.ShapeDtypeStruct((N_EDGES, 2 * EMBED), jnp.float32),
    mesh=_mesh,
    compiler_params=_sc_params,
    scratch_types=[
        pltpu.VMEM((CHUNK,), jnp.int32),
        pltpu.VMEM((CHUNK, EMBED), jnp.float32),
        pltpu.VMEM((TAIL_G,), jnp.int32),
        pltpu.VMEM((TAIL_G, EMBED), jnp.float32),
        pltpu.SemaphoreType.DMA,
    ],
)
def _edge_assemble(emb_hbm, src_hbm, dst_hbm, out_hbm,
                   idx, rows, idxt, rowst, sem):
    cid = lax.axis_index("c")
    sid = lax.axis_index("s")
    base = (sid * NC + cid) * EPW_G

    def sweep(idx_hbm, col0):
        def chunk(i, carry):
            off = base + i * CHUNK
            pltpu.sync_copy(idx_hbm.at[pl.ds(off, CHUNK)], idx)
            pltpu.async_copy(emb_hbm.at[idx], rows, sem).wait()
            pltpu.sync_copy(rows, out_hbm.at[pl.ds(off, CHUNK),
                                             pl.ds(col0, EMBED)])
            return carry

        lax.fori_loop(0, NFULL_G, chunk, 0)
        offt = base + NFULL_G * CHUNK
        pltpu.sync_copy(idx_hbm.at[pl.ds(offt, TAIL_G)], idxt)
        pltpu.async_copy(emb_hbm.at[idxt], rowst, sem).wait()
        pltpu.sync_copy(rowst, out_hbm.at[pl.ds(offt, TAIL_G),
                                          pl.ds(col0, EMBED)])

    sweep(src_hbm, 0)
    sweep(dst_hbm, EMBED)


_BLK_N = 512


def _layer0_tc(xp, aggp, WsT, WnT, b, g, beta):
    def body(x_ref, p_ref, ws_ref, wn_ref, b_ref, g_ref, beta_ref,
             h0_ref, h1_ref, h2_ref, h3_ref, rc_ref):
        p = p_ref[0, 0] + p_ref[0, 1]
        cnt = p[:, 12:13]
        rc = 1.0 / jnp.maximum(cnt, 1.0)
        agg = p[:, 0:12] * rc
        h = (jnp.dot(x_ref[...], ws_ref[...], preferred_element_type=jnp.float32)
             + jnp.dot(agg, wn_ref[...], preferred_element_type=jnp.float32)
             + b_ref[...])
        mu = jnp.mean(h, axis=-1, keepdims=True)
        var = jnp.mean((h - mu) ** 2, axis=-1, keepdims=True)
        h = (h - mu) / jnp.sqrt(var + 1e-5) * g_ref[...] + beta_ref[...]
        h = jnp.maximum(h, 0.0)
        h0_ref[...] = h[:, 0:16]
        h1_ref[...] = h[:, 16:32]
        h2_ref[...] = h[:, 32:48]
        h3_ref[...] = h[:, 48:64]
        rc_ref[...] = rc

    grid = (NP // _BLK_N,)
    hspec = pl.BlockSpec((_BLK_N, 16), lambda i: (i, 0))
    hshape = jax.ShapeDtypeStruct((NP, 16), jnp.float32)
    return pl.pallas_call(
        body,
        grid=grid,
        in_specs=[
            pl.BlockSpec((_BLK_N, NODE_DIM), lambda i: (i, 0)),
            pl.BlockSpec((1, NC, _BLK_N, 16), lambda i: (0, 0, i, 0)),
            pl.BlockSpec((NODE_DIM, HIDDEN), lambda i: (0, 0)),
            pl.BlockSpec((NODE_DIM, HIDDEN), lambda i: (0, 0)),
            pl.BlockSpec((HIDDEN,), lambda i: (0,)),
            pl.BlockSpec((HIDDEN,), lambda i: (0,)),
            pl.BlockSpec((HIDDEN,), lambda i: (0,)),
        ],
        out_specs=[hspec, hspec, hspec, hspec,
                   pl.BlockSpec((_BLK_N, 1), lambda i: (i, 0))],
        out_shape=[hshape, hshape, hshape, hshape,
                   jax.ShapeDtypeStruct((NP, 1), jnp.float32)],
    )(xp, aggp, WsT, WnT, b, g, beta)


def _layer1_tc(h0, h1, h2, h3, aggp, rcnt, WsT, WnT, b, g, beta):
    def body(h0_ref, h1_ref, h2_ref, h3_ref, p_ref, rc_ref, ws_ref, wn_ref,
             b_ref, g_ref, beta_ref, out_ref):
        rc = rc_ref[...]
        h = jnp.concatenate(
            [h0_ref[...], h1_ref[...], h2_ref[...], h3_ref[...]], axis=1)
        agg = jnp.concatenate(
            [p_ref[k, 0] + p_ref[k, 1] for k in range(4)], axis=1) * rc
        e = (jnp.dot(h, ws_ref[...], preferred_element_type=jnp.float32)
             + jnp.dot(agg, wn_ref[...], preferred_element_type=jnp.float32)
             + b_ref[...])
        mu = jnp.mean(e, axis=-1, keepdims=True)
        var = jnp.mean((e - mu) ** 2, axis=-1, keepdims=True)
        e = (e - mu) / jnp.sqrt(var + 1e-5) * g_ref[...] + beta_ref[...]
        out_ref[...] = jnp.maximum(e, 0.0)

    grid = (NP // _BLK_N,)
    hspec = pl.BlockSpec((_BLK_N, 16), lambda i: (i, 0))
    return pl.pallas_call(
        body,
        grid=grid,
        in_specs=[
            hspec, hspec, hspec, hspec,
            pl.BlockSpec((4, NC, _BLK_N, 16), lambda i: (0, 0, i, 0)),
            pl.BlockSpec((_BLK_N, 1), lambda i: (i, 0)),
            pl.BlockSpec((HIDDEN, EMBED), lambda i: (0, 0)),
            pl.BlockSpec((HIDDEN, EMBED), lambda i: (0, 0)),
            pl.BlockSpec((EMBED,), lambda i: (0,)),
            pl.BlockSpec((EMBED,), lambda i: (0,)),
            pl.BlockSpec((EMBED,), lambda i: (0,)),
        ],
        out_specs=pl.BlockSpec((_BLK_N, EMBED), lambda i: (i, 0)),
        out_shape=jax.ShapeDtypeStruct((NP, EMBED), jnp.float32),
    )(h0, h1, h2, h3, aggp, rcnt, WsT, WnT, b, g, beta)


_BLK_E = 2000


def _edge_mlp_tc(hsd, edge_attr, We1T, be1, We2T, be2, Wd1T, bd1, Wd2T, bd2):
    def body(hsd_ref, ea_ref, we1_ref, be1_ref, we2_ref, be2_ref,
             wd1_ref, bd1_ref, wd2_ref, bd2_ref, rec_ref, er_ref):
        er = jnp.concatenate([hsd_ref[...], ea_ref[...]], axis=1)
        er_ref[...] = er
        l1 = jnp.maximum(
            jnp.dot(er, we1_ref[...], preferred_element_type=jnp.float32)
            + be1_ref[...], 0.0)
        lat = jnp.maximum(
            jnp.dot(l1, we2_ref[...], preferred_element_type=jnp.float32)
            + be2_ref[...], 0.0)
        d1 = jnp.maximum(
            jnp.dot(lat, wd1_ref[...], preferred_element_type=jnp.float32)
            + bd1_ref[...], 0.0)
        rec_ref[...] = (jnp.dot(d1, wd2_ref[...], preferred_element_type=jnp.float32)
                        + bd2_ref[...])

    grid = (N_EDGES // _BLK_E,)
    full = lambda shape: pl.BlockSpec(shape, lambda i: tuple(0 for _ in shape))
    return pl.pallas_call(
        body,
        grid=grid,
        in_specs=[
            pl.BlockSpec((_BLK_E, 2 * EMBED), lambda i: (i, 0)),
            pl.BlockSpec((_BLK_E, 10), lambda i: (i, 0)),
            full((EDGE_REPR, HIDDEN)),
            full((HIDDEN,)),
            full((HIDDEN, EMBED)),
            full((EMBED,)),
            full((EMBED, HIDDEN)),
            full((HIDDEN,)),
            full((HIDDEN, EDGE_REPR)),
            full((EDGE_REPR,)),
        ],
        out_specs=[
            pl.BlockSpec((_BLK_E, EDGE_REPR), lambda i: (i, 0)),
            pl.BlockSpec((_BLK_E, EDGE_REPR), lambda i: (i, 0)),
        ],
        out_shape=[
            jax.ShapeDtypeStruct((N_EDGES, EDGE_REPR), jnp.float32),
            jax.ShapeDtypeStruct((N_EDGES, EDGE_REPR), jnp.float32),
        ],
    )(hsd, edge_attr, We1T, be1, We2T, be2, Wd1T, bd1, Wd2T, bd2)


_segsum1 = _make_segsum(1)
_segsum4 = _make_segsum(4)


def kernel(node_feats, edge_attr, W_self0, W_neigh0, b0, g0, beta0,
           W_self1, W_neigh1, b1, g1, beta1, We1, be1, We2, be2,
           Wd1, bd1, Wd2, bd2, edge_index, src_nodes, dst_nodes):
    s = edge_index[0]
    d = edge_index[1]

    npad = E_PAD - N_EDGES
    pad_ids = N_NODES + (jnp.arange(npad, dtype=jnp.int32) % N_DUMMY)
    s_pad = jnp.concatenate([s, pad_ids])
    d_pad = jnp.concatenate([d, pad_ids])

    x16 = jnp.concatenate(
        [node_feats,
         jnp.ones((N_NODES, 1), jnp.float32),
         jnp.zeros((N_NODES, 3), jnp.float32)], axis=1)
    x16 = jnp.pad(x16, ((0, NP - N_NODES), (0, 0)))
    xp = jnp.pad(node_feats, ((0, NP - N_NODES), (0, 0)))

    zeros16 = jnp.zeros((RPT, 16), jnp.float32)

    agg0p = _segsum1(x16, s_pad, d_pad, zeros16)
    h0, h1, h2, h3, rcnt = _layer0_tc(
        xp, agg0p, W_self0.T, W_neigh0.T, b0, g0, beta0)

    agg1p = _segsum4(h0, h1, h2, h3, s_pad, d_pad, zeros16)
    emb = _layer1_tc(h0, h1, h2, h3, agg1p, rcnt,
                     W_self1.T, W_neigh1.T, b1, g1, beta1)

    hsd = _edge_assemble(emb, src_nodes, dst_nodes)
    recon, edge_rep = _edge_mlp_tc(
        hsd, edge_attr, We1.T, be1, We2.T, be2, Wd1.T, bd1, Wd2.T, bd2)
    return (recon, edge_rep)


# speedup vs baseline: 3.9170x; 1.5496x over previous
import functools

import jax
import jax.numpy as jnp
from jax import lax
from jax.experimental import pallas as pl
from jax.experimental.pallas import tpu as pltpu
from jax.experimental.pallas import tpu_sc as plsc

N_NODES = 50000
N_EDGES = 800000
NODE_DIM = 12
HIDDEN = 64
EMBED = 32
EDGE_REPR = 74

NC = 2
NS = 16
NW = NC * NS

NP = 50176
RPT = NP // NS
N_DUMMY = 176

E_PAD = 819200
EPW_SEG = E_PAD // NW
CHUNK = 1024
NCHUNK_SEG = EPW_SEG // CHUNK

EPW_G = N_EDGES // NW
NFULL_G = EPW_G // CHUNK
TAIL_G = EPW_G - NFULL_G * CHUNK

_mesh = plsc.VectorSubcoreMesh(core_axis_name="c", subcore_axis_name="s")
_sc_params = pltpu.CompilerParams(use_tc_tiling_on_sc=False)


def _make_segsum(n_tables):
    @functools.partial(
        pl.kernel,
        out_type=jax.ShapeDtypeStruct((n_tables, NC, NP, 16), jnp.float32),
        mesh=_mesh,
        compiler_params=_sc_params,
        scratch_types=[
            pltpu.VMEM_SHARED((NP, 16), jnp.float32),
            pltpu.VMEM((RPT, 16), jnp.float32),
            pltpu.VMEM((CHUNK,), jnp.int32),
            pltpu.VMEM((CHUNK,), jnp.int32),
            pltpu.VMEM((CHUNK, 16), jnp.float32),
            pltpu.SemaphoreType.DMA,
        ],
    )
    def seg_kernel(*refs):
        tabs = refs[:n_tables]
        s_hbm, d_hbm, zeros_hbm, out_hbm = refs[n_tables:n_tables + 4]
        accum, stage, sidx, didx, rows, sem = refs[n_tables + 4:]
        cid = lax.axis_index("c")
        sid = lax.axis_index("s")
        base = (sid * NC + cid) * EPW_SEG
        for k in range(n_tables):
            pltpu.sync_copy(zeros_hbm, stage)
            pltpu.sync_copy(stage, accum.at[pl.ds(sid * RPT, RPT)])
            plsc.subcore_barrier()

            def chunk(i, carry):
                off = base + i * CHUNK
                pltpu.sync_copy(s_hbm.at[pl.ds(off, CHUNK)], sidx)
                pltpu.sync_copy(d_hbm.at[pl.ds(off, CHUNK)], didx)
                pltpu.async_copy(tabs[k].at[sidx], rows, sem).wait()
                pltpu.sync_copy(rows, accum.at[didx], add=True)
                return carry

            lax.fori_loop(0, NCHUNK_SEG, chunk, 0)
            plsc.subcore_barrier()
            pltpu.sync_copy(accum.at[pl.ds(sid * RPT, RPT)], stage)
            pltpu.sync_copy(stage, out_hbm.at[k, cid, pl.ds(sid * RPT, RPT)])
            plsc.subcore_barrier()

    return seg_kernel


@functools.partial(
    pl.kernel,
    out_type=jax.ShapeDtypeStruct((N_EDGES, 2 * EMBED), jnp.float32),
    mesh=_mesh,
    compiler_params=_sc_params,
    scratch_types=[
        pltpu.VMEM((CHUNK,), jnp.int32),
        pltpu.VMEM((CHUNK, EMBED), jnp.float32),
        pltpu.VMEM((TAIL_G,), jnp.int32),
        pltpu.VMEM((TAIL_G, EMBED), jnp.float32),
        pltpu.SemaphoreType.DMA,
    ],
)
def _edge_assemble(emb_hbm, src_hbm, dst_hbm, out_hbm,
                   idx, rows, idxt, rowst, sem):
    cid = lax.axis_index("c")
    sid = lax.axis_index("s")
    base = (sid * NC + cid) * EPW_G

    def sweep(idx_hbm, col0):
        def chunk(i, carry):
            off = base + i * CHUNK
            pltpu.sync_copy(idx_hbm.at[pl.ds(off, CHUNK)], idx)
            pltpu.async_copy(emb_hbm.at[idx], rows, sem).wait()
            pltpu.sync_copy(rows, out_hbm.at[pl.ds(off, CHUNK),
                                             pl.ds(col0, EMBED)])
            return carry

        lax.fori_loop(0, NFULL_G, chunk, 0)
        offt = base + NFULL_G * CHUNK
        pltpu.sync_copy(idx_hbm.at[pl.ds(offt, TAIL_G)], idxt)
        pltpu.async_copy(emb_hbm.at[idxt], rowst, sem).wait()
        pltpu.sync_copy(rowst, out_hbm.at[pl.ds(offt, TAIL_G),
                                          pl.ds(col0, EMBED)])

    sweep(src_hbm, 0)
    sweep(dst_hbm, EMBED)


_BLK_N = 512


def _layer0_tc(xp, aggp, WsT, WnT, b, g, beta):
    def body(x_ref, p_ref, ws_ref, wn_ref, b_ref, g_ref, beta_ref,
             h0_ref, h1_ref, h2_ref, h3_ref, rc_ref):
        p = p_ref[0, 0] + p_ref[0, 1]
        cnt = p[:, 12:13]
        rc = 1.0 / jnp.maximum(cnt, 1.0)
        agg = p[:, 0:12] * rc
        h = (jnp.dot(x_ref[...], ws_ref[...], preferred_element_type=jnp.float32)
             + jnp.dot(agg, wn_ref[...], preferred_element_type=jnp.float32)
             + b_ref[...])
        mu = jnp.mean(h, axis=-1, keepdims=True)
        var = jnp.mean((h - mu) ** 2, axis=-1, keepdims=True)
        h = (h - mu) / jnp.sqrt(var + 1e-5) * g_ref[...] + beta_ref[...]
        h = jnp.maximum(h, 0.0)
        h0_ref[...] = h[:, 0:16]
        h1_ref[...] = h[:, 16:32]
        h2_ref[...] = h[:, 32:48]
        h3_ref[...] = h[:, 48:64]
        rc_ref[...] = rc

    grid = (NP // _BLK_N,)
    hspec = pl.BlockSpec((_BLK_N, 16), lambda i: (i, 0))
    hshape = jax.ShapeDtypeStruct((NP, 16), jnp.float32)
    return pl.pallas_call(
        body,
        grid=grid,
        in_specs=[
            pl.BlockSpec((_BLK_N, NODE_DIM), lambda i: (i, 0)),
            pl.BlockSpec((1, NC, _BLK_N, 16), lambda i: (0, 0, i, 0)),
            pl.BlockSpec((NODE_DIM, HIDDEN), lambda i: (0, 0)),
            pl.BlockSpec((NODE_DIM, HIDDEN), lambda i: (0, 0)),
            pl.BlockSpec((HIDDEN,), lambda i: (0,)),
            pl.BlockSpec((HIDDEN,), lambda i: (0,)),
            pl.BlockSpec((HIDDEN,), lambda i: (0,)),
        ],
        out_specs=[hspec, hspec, hspec, hspec,
                   pl.BlockSpec((_BLK_N, 1), lambda i: (i, 0))],
        out_shape=[hshape, hshape, hshape, hshape,
                   jax.ShapeDtypeStruct((NP, 1), jnp.float32)],
    )(xp, aggp, WsT, WnT, b, g, beta)


def _layer1_tc(h0, h1, h2, h3, aggp, rcnt, WsT, WnT, b, g, beta):
    def body(h0_ref, h1_ref, h2_ref, h3_ref, p_ref, rc_ref, ws_ref, wn_ref,
             b_ref, g_ref, beta_ref, out_ref):
        rc = rc_ref[...]
        h = jnp.concatenate(
            [h0_ref[...], h1_ref[...], h2_ref[...], h3_ref[...]], axis=1)
        agg = jnp.concatenate(
            [p_ref[k, 0] + p_ref[k, 1] for k in range(4)], axis=1) * rc
        e = (jnp.dot(h, ws_ref[...], preferred_element_type=jnp.float32)
             + jnp.dot(agg, wn_ref[...], preferred_element_type=jnp.float32)
             + b_ref[...])
        mu = jnp.mean(e, axis=-1, keepdims=True)
        var = jnp.mean((e - mu) ** 2, axis=-1, keepdims=True)
        e = (e - mu) / jnp.sqrt(var + 1e-5) * g_ref[...] + beta_ref[...]
        out_ref[...] = jnp.maximum(e, 0.0)

    grid = (NP // _BLK_N,)
    hspec = pl.BlockSpec((_BLK_N, 16), lambda i: (i, 0))
    return pl.pallas_call(
        body,
        grid=grid,
        in_specs=[
            hspec, hspec, hspec, hspec,
            pl.BlockSpec((4, NC, _BLK_N, 16), lambda i: (0, 0, i, 0)),
            pl.BlockSpec((_BLK_N, 1), lambda i: (i, 0)),
            pl.BlockSpec((HIDDEN, EMBED), lambda i: (0, 0)),
            pl.BlockSpec((HIDDEN, EMBED), lambda i: (0, 0)),
            pl.BlockSpec((EMBED,), lambda i: (0,)),
            pl.BlockSpec((EMBED,), lambda i: (0,)),
            pl.BlockSpec((EMBED,), lambda i: (0,)),
        ],
        out_specs=pl.BlockSpec((_BLK_N, EMBED), lambda i: (i, 0)),
        out_shape=jax.ShapeDtypeStruct((NP, EMBED), jnp.float32),
    )(h0, h1, h2, h3, aggp, rcnt, WsT, WnT, b, g, beta)


_BLK_E = 2000


def _edge_mlp_tc(hsd, edge_attr, We1T, be1, We2T, be2, Wd1T, bd1, Wd2T, bd2):
    def body(hsd_ref, ea_ref, we1_ref, be1_ref, we2_ref, be2_ref,
             wd1_ref, bd1_ref, wd2_ref, bd2_ref, rec_ref, er_ref):
        er = jnp.concatenate([hsd_ref[...], ea_ref[...]], axis=1)
        er_ref[...] = er
        l1 = jnp.maximum(
            jnp.dot(er, we1_ref[...], preferred_element_type=jnp.float32)
            + be1_ref[...], 0.0)
        lat = jnp.maximum(
            jnp.dot(l1, we2_ref[...], preferred_element_type=jnp.float32)
            + be2_ref[...], 0.0)
        d1 = jnp.maximum(
            jnp.dot(lat, wd1_ref[...], preferred_element_type=jnp.float32)
            + bd1_ref[...], 0.0)
        rec_ref[...] = (jnp.dot(d1, wd2_ref[...], preferred_element_type=jnp.float32)
                        + bd2_ref[...])

    grid = (N_EDGES // _BLK_E,)
    full = lambda shape: pl.BlockSpec(shape, lambda i: tuple(0 for _ in shape))
    return pl.pallas_call(
        body,
        grid=grid,
        in_specs=[
            pl.BlockSpec((_BLK_E, 2 * EMBED), lambda i: (i, 0)),
            pl.BlockSpec((_BLK_E, 10), lambda i: (i, 0)),
            full((EDGE_REPR, HIDDEN)),
            full((HIDDEN,)),
            full((HIDDEN, EMBED)),
            full((EMBED,)),
            full((EMBED, HIDDEN)),
            full((HIDDEN,)),
            full((HIDDEN, EDGE_REPR)),
            full((EDGE_REPR,)),
        ],
        out_specs=[
            pl.BlockSpec((_BLK_E, EDGE_REPR), lambda i: (i, 0)),
            pl.BlockSpec((_BLK_E, EDGE_REPR), lambda i: (i, 0)),
        ],
        out_shape=[
            jax.ShapeDtypeStruct((N_EDGES, EDGE_REPR), jnp.float32),
            jax.ShapeDtypeStruct((N_EDGES, EDGE_REPR), jnp.float32),
        ],
    )(hsd, edge_attr, We1T, be1, We2T, be2, Wd1T, bd1, Wd2T, bd2)


_segsum1 = _make_segsum(1)
_segsum4 = _make_segsum(4)


def kernel(node_feats, edge_attr, W_self0, W_neigh0, b0, g0, beta0,
           W_self1, W_neigh1, b1, g1, beta1, We1, be1, We2, be2,
           Wd1, bd1, Wd2, bd2, edge_index, src_nodes, dst_nodes):
    s = edge_index[0]
    d = edge_index[1]

    npad = E_PAD - N_EDGES
    pad_ids = N_NODES + (jnp.arange(npad, dtype=jnp.int32) % N_DUMMY)
    s_pad = jnp.concatenate([s, pad_ids])
    d_pad = jnp.concatenate([d, pad_ids])

    x16 = jnp.concatenate(
        [node_feats,
         jnp.ones((N_NODES, 1), jnp.float32),
         jnp.zeros((N_NODES, 3), jnp.float32)], axis=1)
    x16 = jnp.pad(x16, ((0, NP - N_NODES), (0, 0)))
    xp = jnp.pad(node_feats, ((0, NP - N_NODES), (0, 0)))

    zeros16 = jnp.zeros((RPT, 16), jnp.float32)

    agg0p = _segsum1(x16, s_pad, d_pad, zeros16)
    h0, h1, h2, h3, rcnt = _layer0_tc(
        xp, agg0p, W_self0.T, W_neigh0.T, b0, g0, beta0)

    agg1p = _segsum4(h0, h1, h2, h3, s_pad, d_pad, zeros16)
    emb = _layer1_tc(h0, h1, h2, h3, agg1p, rcnt,
                     W_self1.T, W_neigh1.T, b1, g1, beta1)

    hsd = _edge_assemble(emb, src_nodes, dst_nodes)
    recon, edge_rep = _edge_mlp_tc(
        hsd, edge_attr, We1.T, be1, We2.T, be2, Wd1.T, bd1, Wd2.T, bd2)
    return (recon, edge_rep)


# speedup vs baseline: 4.1397x; 1.0569x over previous
import functools

import jax
import jax.numpy as jnp
from jax import lax
from jax.experimental import pallas as pl
from jax.experimental.pallas import tpu as pltpu
from jax.experimental.pallas import tpu_sc as plsc

N_NODES = 50000
N_EDGES = 800000
NODE_DIM = 12
HIDDEN = 64
EMBED = 32
EDGE_REPR = 74

NC = 2
NS = 16
NW = NC * NS

NP = 50176
RPT = NP // NS
N_DUMMY = 176

E_PAD = 819200
EPW_SEG = E_PAD // NW
CHUNK = 1024
CHUNK_SEG = 800
NCHUNK_SEG = EPW_SEG // CHUNK_SEG

EPW_G = N_EDGES // NW
NFULL_G = EPW_G // CHUNK
TAIL_G = EPW_G - NFULL_G * CHUNK

_mesh = plsc.VectorSubcoreMesh(core_axis_name="c", subcore_axis_name="s")
_sc_params = pltpu.CompilerParams(use_tc_tiling_on_sc=False)


def _make_segsum(n_tables):
    @functools.partial(
        pl.kernel,
        out_type=jax.ShapeDtypeStruct((n_tables, NC, NP, 16), jnp.float32),
        mesh=_mesh,
        compiler_params=_sc_params,
        scratch_types=[
            pltpu.VMEM_SHARED((NP, 16), jnp.float32),
            pltpu.VMEM((RPT, 16), jnp.float32),
            pltpu.VMEM((CHUNK_SEG,), jnp.int32),
            pltpu.VMEM((CHUNK_SEG,), jnp.int32),
            pltpu.VMEM((CHUNK_SEG,), jnp.int32),
            pltpu.VMEM((CHUNK_SEG,), jnp.int32),
            pltpu.VMEM((CHUNK_SEG, 16), jnp.float32),
            pltpu.VMEM((CHUNK_SEG, 16), jnp.float32),
            pltpu.SemaphoreType.DMA,
            pltpu.SemaphoreType.DMA,
        ],
    )
    def seg_kernel(*refs):
        tabs = refs[:n_tables]
        s_hbm, d_hbm, zeros_hbm, out_hbm = refs[n_tables:n_tables + 4]
        accum, stage = refs[n_tables + 4:n_tables + 6]
        sidx = refs[n_tables + 6:n_tables + 8]
        didx = refs[n_tables + 8:n_tables + 10]
        rows = refs[n_tables + 10:n_tables + 12]
        gsem = refs[n_tables + 12:n_tables + 14]
        cid = lax.axis_index("c")
        sid = lax.axis_index("s")
        base = (sid * NC + cid) * EPW_SEG
        for k in range(n_tables):
            pltpu.sync_copy(zeros_hbm, stage)
            pltpu.sync_copy(stage, accum.at[pl.ds(sid * RPT, RPT)])
            plsc.subcore_barrier()

            def fire_gather(b, i):
                off = base + i * CHUNK_SEG
                pltpu.sync_copy(s_hbm.at[pl.ds(off, CHUNK_SEG)], sidx[b])
                pltpu.sync_copy(d_hbm.at[pl.ds(off, CHUNK_SEG)], didx[b])
                pltpu.async_copy(tabs[k].at[sidx[b]], rows[b], gsem[b])

            fire_gather(0, 0)
            fire_gather(1, 1)

            @pl.loop(0, NCHUNK_SEG, step=2)
            def _pair(g):
                for b in range(2):
                    i = g + b

                    @pl.when(i < NCHUNK_SEG)
                    def _():
                        pltpu.make_async_copy(
                            tabs[k].at[sidx[b]], rows[b], gsem[b]).wait()
                        pltpu.sync_copy(rows[b], accum.at[didx[b]], add=True)

                        @pl.when(i + 2 < NCHUNK_SEG)
                        def _():
                            fire_gather(b, i + 2)

            plsc.subcore_barrier()
            pltpu.sync_copy(accum.at[pl.ds(sid * RPT, RPT)], stage)
            pltpu.sync_copy(stage, out_hbm.at[k, cid, pl.ds(sid * RPT, RPT)])
            plsc.subcore_barrier()

    return seg_kernel


@functools.partial(
    pl.kernel,
    out_type=jax.ShapeDtypeStruct((N_EDGES, 2 * EMBED), jnp.float32),
    mesh=_mesh,
    compiler_params=_sc_params,
    scratch_types=[
        pltpu.VMEM((CHUNK,), jnp.int32),
        pltpu.VMEM((CHUNK,), jnp.int32),
        pltpu.VMEM((CHUNK, EMBED), jnp.float32),
        pltpu.VMEM((CHUNK, EMBED), jnp.float32),
        pltpu.VMEM((TAIL_G,), jnp.int32),
        pltpu.VMEM((TAIL_G, EMBED), jnp.float32),
        pltpu.SemaphoreType.DMA,
        pltpu.SemaphoreType.DMA,
        pltpu.SemaphoreType.DMA,
        pltpu.SemaphoreType.DMA,
    ],
)
def _edge_assemble(emb_hbm, src_hbm, dst_hbm, out_hbm,
                   idx0, idx1, rows0, rows1, idxt, rowst,
                   gsem0, gsem1, wsem0, wsem1):
    idx = (idx0, idx1)
    rows = (rows0, rows1)
    gsem = (gsem0, gsem1)
    wsem = (wsem0, wsem1)
    cid = lax.axis_index("c")
    sid = lax.axis_index("s")
    base = (sid * NC + cid) * EPW_G

    def sweep(idx_hbm, col0):
        def out_slice(b, i):
            off = base + i * CHUNK
            return out_hbm.at[pl.ds(off, CHUNK), pl.ds(col0, EMBED)]

        def fire_gather(b, i):
            off = base + i * CHUNK
            pltpu.sync_copy(idx_hbm.at[pl.ds(off, CHUNK)], idx[b])
            pltpu.async_copy(emb_hbm.at[idx[b]], rows[b], gsem[b])

        fire_gather(0, 0)
        fire_gather(1, 1)

        @pl.loop(0, NFULL_G, step=2)
        def _pair(g):
            for b in range(2):
                i = g + b

                @pl.when(i < NFULL_G)
                def _():
                    pltpu.make_async_copy(
                        emb_hbm.at[idx[b]], rows[b], gsem[b]).wait()
                    pltpu.async_copy(rows[b], out_slice(b, i), wsem[b])

                    @pl.when(i + 2 < NFULL_G)
                    def _():
                        pltpu.make_async_copy(
                            rows[b], out_slice(b, i), wsem[b]).wait()
                        fire_gather(b, i + 2)

        for b in range(2):
            pltpu.make_async_copy(rows[b], out_slice(b, 0), wsem[b]).wait()
        offt = base + NFULL_G * CHUNK
        pltpu.sync_copy(idx_hbm.at[pl.ds(offt, TAIL_G)], idxt)
        pltpu.async_copy(emb_hbm.at[idxt], rowst, gsem[0]).wait()
        pltpu.sync_copy(rowst, out_hbm.at[pl.ds(offt, TAIL_G),
                                          pl.ds(col0, EMBED)])

    sweep(src_hbm, 0)
    sweep(dst_hbm, EMBED)


_BLK_N = 512


def _layer0_tc(xp, aggp, WsT, WnT, b, g, beta):
    def body(x_ref, p_ref, ws_ref, wn_ref, b_ref, g_ref, beta_ref,
             h0_ref, h1_ref, h2_ref, h3_ref, rc_ref):
        p = p_ref[0, 0] + p_ref[0, 1]
        cnt = p[:, 12:13]
        rc = 1.0 / jnp.maximum(cnt, 1.0)
        agg = p[:, 0:12] * rc
        h = (jnp.dot(x_ref[...], ws_ref[...], preferred_element_type=jnp.float32)
             + jnp.dot(agg, wn_ref[...], preferred_element_type=jnp.float32)
             + b_ref[...])
        mu = jnp.mean(h, axis=-1, keepdims=True)
        var = jnp.mean((h - mu) ** 2, axis=-1, keepdims=True)
        h = (h - mu) / jnp.sqrt(var + 1e-5) * g_ref[...] + beta_ref[...]
        h = jnp.maximum(h, 0.0)
        h0_ref[...] = h[:, 0:16]
        h1_ref[...] = h[:, 16:32]
        h2_ref[...] = h[:, 32:48]
        h3_ref[...] = h[:, 48:64]
        rc_ref[...] = rc

    grid = (NP // _BLK_N,)
    hspec = pl.BlockSpec((_BLK_N, 16), lambda i: (i, 0))
    hshape = jax.ShapeDtypeStruct((NP, 16), jnp.float32)
    return pl.pallas_call(
        body,
        grid=grid,
        in_specs=[
            pl.BlockSpec((_BLK_N, NODE_DIM), lambda i: (i, 0)),
            pl.BlockSpec((1, NC, _BLK_N, 16), lambda i: (0, 0, i, 0)),
            pl.BlockSpec((NODE_DIM, HIDDEN), lambda i: (0, 0)),
            pl.BlockSpec((NODE_DIM, HIDDEN), lambda i: (0, 0)),
            pl.BlockSpec((HIDDEN,), lambda i: (0,)),
            pl.BlockSpec((HIDDEN,), lambda i: (0,)),
            pl.BlockSpec((HIDDEN,), lambda i: (0,)),
        ],
        out_specs=[hspec, hspec, hspec, hspec,
                   pl.BlockSpec((_BLK_N, 1), lambda i: (i, 0))],
        out_shape=[hshape, hshape, hshape, hshape,
                   jax.ShapeDtypeStruct((NP, 1), jnp.float32)],
    )(xp, aggp, WsT, WnT, b, g, beta)


def _layer1_tc(h0, h1, h2, h3, aggp, rcnt, WsT, WnT, b, g, beta):
    def body(h0_ref, h1_ref, h2_ref, h3_ref, p_ref, rc_ref, ws_ref, wn_ref,
             b_ref, g_ref, beta_ref, out_ref):
        rc = rc_ref[...]
        h = jnp.concatenate(
            [h0_ref[...], h1_ref[...], h2_ref[...], h3_ref[...]], axis=1)
        agg = jnp.concatenate(
            [p_ref[k, 0] + p_ref[k, 1] for k in range(4)], axis=1) * rc
        e = (jnp.dot(h, ws_ref[...], preferred_element_type=jnp.float32)
             + jnp.dot(agg, wn_ref[...], preferred_element_type=jnp.float32)
             + b_ref[...])
        mu = jnp.mean(e, axis=-1, keepdims=True)
        var = jnp.mean((e - mu) ** 2, axis=-1, keepdims=True)
        e = (e - mu) / jnp.sqrt(var + 1e-5) * g_ref[...] + beta_ref[...]
        out_ref[...] = jnp.maximum(e, 0.0)

    grid = (NP // _BLK_N,)
    hspec = pl.BlockSpec((_BLK_N, 16), lambda i: (i, 0))
    return pl.pallas_call(
        body,
        grid=grid,
        in_specs=[
            hspec, hspec, hspec, hspec,
            pl.BlockSpec((4, NC, _BLK_N, 16), lambda i: (0, 0, i, 0)),
            pl.BlockSpec((_BLK_N, 1), lambda i: (i, 0)),
            pl.BlockSpec((HIDDEN, EMBED), lambda i: (0, 0)),
            pl.BlockSpec((HIDDEN, EMBED), lambda i: (0, 0)),
            pl.BlockSpec((EMBED,), lambda i: (0,)),
            pl.BlockSpec((EMBED,), lambda i: (0,)),
            pl.BlockSpec((EMBED,), lambda i: (0,)),
        ],
        out_specs=pl.BlockSpec((_BLK_N, EMBED), lambda i: (i, 0)),
        out_shape=jax.ShapeDtypeStruct((NP, EMBED), jnp.float32),
    )(h0, h1, h2, h3, aggp, rcnt, WsT, WnT, b, g, beta)


_BLK_E = 2000


def _edge_mlp_tc(hsd, edge_attr, We1T, be1, We2T, be2, Wd1T, bd1, Wd2T, bd2):
    def body(hsd_ref, ea_ref, we1_ref, be1_ref, we2_ref, be2_ref,
             wd1_ref, bd1_ref, wd2_ref, bd2_ref, rec_ref, er_ref):
        er = jnp.concatenate([hsd_ref[...], ea_ref[...]], axis=1)
        er_ref[...] = er
        l1 = jnp.maximum(
            jnp.dot(er, we1_ref[...], preferred_element_type=jnp.float32)
            + be1_ref[...], 0.0)
        lat = jnp.maximum(
            jnp.dot(l1, we2_ref[...], preferred_element_type=jnp.float32)
            + be2_ref[...], 0.0)
        d1 = jnp.maximum(
            jnp.dot(lat, wd1_ref[...], preferred_element_type=jnp.float32)
            + bd1_ref[...], 0.0)
        rec_ref[...] = (jnp.dot(d1, wd2_ref[...], preferred_element_type=jnp.float32)
                        + bd2_ref[...])

    grid = (N_EDGES // _BLK_E,)
    full = lambda shape: pl.BlockSpec(shape, lambda i: tuple(0 for _ in shape))
    return pl.pallas_call(
        body,
        grid=grid,
        in_specs=[
            pl.BlockSpec((_BLK_E, 2 * EMBED), lambda i: (i, 0)),
            pl.BlockSpec((_BLK_E, 10), lambda i: (i, 0)),
            full((EDGE_REPR, HIDDEN)),
            full((HIDDEN,)),
            full((HIDDEN, EMBED)),
            full((EMBED,)),
            full((EMBED, HIDDEN)),
            full((HIDDEN,)),
            full((HIDDEN, EDGE_REPR)),
            full((EDGE_REPR,)),
        ],
        out_specs=[
            pl.BlockSpec((_BLK_E, EDGE_REPR), lambda i: (i, 0)),
            pl.BlockSpec((_BLK_E, EDGE_REPR), lambda i: (i, 0)),
        ],
        out_shape=[
            jax.ShapeDtypeStruct((N_EDGES, EDGE_REPR), jnp.float32),
            jax.ShapeDtypeStruct((N_EDGES, EDGE_REPR), jnp.float32),
        ],
    )(hsd, edge_attr, We1T, be1, We2T, be2, Wd1T, bd1, Wd2T, bd2)


_segsum1 = _make_segsum(1)
_segsum4 = _make_segsum(4)


def kernel(node_feats, edge_attr, W_self0, W_neigh0, b0, g0, beta0,
           W_self1, W_neigh1, b1, g1, beta1, We1, be1, We2, be2,
           Wd1, bd1, Wd2, bd2, edge_index, src_nodes, dst_nodes):
    s = edge_index[0]
    d = edge_index[1]

    npad = E_PAD - N_EDGES
    pad_ids = N_NODES + (jnp.arange(npad, dtype=jnp.int32) % N_DUMMY)
    s_pad = jnp.concatenate([s, pad_ids])
    d_pad = jnp.concatenate([d, pad_ids])

    x16 = jnp.concatenate(
        [node_feats,
         jnp.ones((N_NODES, 1), jnp.float32),
         jnp.zeros((N_NODES, 3), jnp.float32)], axis=1)
    x16 = jnp.pad(x16, ((0, NP - N_NODES), (0, 0)))
    xp = jnp.pad(node_feats, ((0, NP - N_NODES), (0, 0)))

    zeros16 = jnp.zeros((RPT, 16), jnp.float32)

    agg0p = _segsum1(x16, s_pad, d_pad, zeros16)
    h0, h1, h2, h3, rcnt = _layer0_tc(
        xp, agg0p, W_self0.T, W_neigh0.T, b0, g0, beta0)

    agg1p = _segsum4(h0, h1, h2, h3, s_pad, d_pad, zeros16)
    emb = _layer1_tc(h0, h1, h2, h3, agg1p, rcnt,
                     W_self1.T, W_neigh1.T, b1, g1, beta1)

    hsd = _edge_assemble(emb, src_nodes, dst_nodes)
    recon, edge_rep = _edge_mlp_tc(
        hsd, edge_attr, We1.T, be1, We2.T, be2, Wd1.T, bd1, Wd2.T, bd2)
    return (recon, edge_rep)


# speedup vs baseline: 6.2921x; 1.5199x over previous
import functools

import jax
import jax.numpy as jnp
from jax import lax
from jax.experimental import pallas as pl
from jax.experimental.pallas import tpu as pltpu
from jax.experimental.pallas import tpu_sc as plsc

N_NODES = 50000
N_EDGES = 800000
NODE_DIM = 12
HIDDEN = 64
EMBED = 32
EDGE_REPR = 74

NC = 2
NS = 16
NW = NC * NS

NP = 50176
RPT = NP // NS
N_DUMMY = 176

E_PAD = 819200
EPW_SEG = E_PAD // NW
CHUNK = 1024
CHUNK_SEG = 800
NCHUNK_SEG = EPW_SEG // CHUNK_SEG

EPW_G = N_EDGES // NW
NFULL_G = EPW_G // CHUNK
TAIL_G = EPW_G - NFULL_G * CHUNK

_mesh = plsc.VectorSubcoreMesh(core_axis_name="c", subcore_axis_name="s")
_sc_params = pltpu.CompilerParams(use_tc_tiling_on_sc=False)


def _make_segsum(n_tables):
    @functools.partial(
        pl.kernel,
        out_type=jax.ShapeDtypeStruct((n_tables, NC, NP, 16), jnp.float32),
        mesh=_mesh,
        compiler_params=_sc_params,
        scratch_types=[
            pltpu.VMEM_SHARED((NP, 16), jnp.float32),
            pltpu.VMEM((RPT, 16), jnp.float32),
            pltpu.VMEM((CHUNK_SEG,), jnp.int32),
            pltpu.VMEM((CHUNK_SEG,), jnp.int32),
            pltpu.VMEM((CHUNK_SEG,), jnp.int32),
            pltpu.VMEM((CHUNK_SEG,), jnp.int32),
            pltpu.VMEM((CHUNK_SEG, 16), jnp.float32),
            pltpu.VMEM((CHUNK_SEG, 16), jnp.float32),
            pltpu.SemaphoreType.DMA,
            pltpu.SemaphoreType.DMA,
        ],
    )
    def seg_kernel(*refs):
        tabs = refs[:n_tables]
        s_hbm, d_hbm, zeros_hbm, out_hbm = refs[n_tables:n_tables + 4]
        accum, stage = refs[n_tables + 4:n_tables + 6]
        sidx = refs[n_tables + 6:n_tables + 8]
        didx = refs[n_tables + 8:n_tables + 10]
        rows = refs[n_tables + 10:n_tables + 12]
        gsem = refs[n_tables + 12:n_tables + 14]
        cid = lax.axis_index("c")
        sid = lax.axis_index("s")
        base = (sid * NC + cid) * EPW_SEG
        for k in range(n_tables):
            pltpu.sync_copy(zeros_hbm, stage)
            pltpu.sync_copy(stage, accum.at[pl.ds(sid * RPT, RPT)])
            plsc.subcore_barrier()

            def fire_gather(b, i):
                off = base + i * CHUNK_SEG
                pltpu.sync_copy(s_hbm.at[pl.ds(off, CHUNK_SEG)], sidx[b])
                pltpu.sync_copy(d_hbm.at[pl.ds(off, CHUNK_SEG)], didx[b])
                pltpu.async_copy(tabs[k].at[sidx[b]], rows[b], gsem[b])

            fire_gather(0, 0)
            fire_gather(1, 1)

            @pl.loop(0, NCHUNK_SEG, step=2)
            def _pair(g):
                for b in range(2):
                    i = g + b

                    @pl.when(i < NCHUNK_SEG)
                    def _():
                        pltpu.make_async_copy(
                            tabs[k].at[sidx[b]], rows[b], gsem[b]).wait()
                        pltpu.sync_copy(rows[b], accum.at[didx[b]], add=True)

                        @pl.when(i + 2 < NCHUNK_SEG)
                        def _():
                            fire_gather(b, i + 2)

            plsc.subcore_barrier()
            pltpu.sync_copy(accum.at[pl.ds(sid * RPT, RPT)], stage)
            pltpu.sync_copy(stage, out_hbm.at[k, cid, pl.ds(sid * RPT, RPT)])
            plsc.subcore_barrier()

    return seg_kernel


@functools.partial(
    pl.kernel,
    out_type=jax.ShapeDtypeStruct((N_EDGES, 2 * EMBED), jnp.float32),
    mesh=_mesh,
    compiler_params=_sc_params,
    scratch_types=[
        pltpu.VMEM((CHUNK,), jnp.int32),
        pltpu.VMEM((CHUNK,), jnp.int32),
        pltpu.VMEM((CHUNK, EMBED), jnp.float32),
        pltpu.VMEM((CHUNK, EMBED), jnp.float32),
        pltpu.VMEM((TAIL_G,), jnp.int32),
        pltpu.VMEM((TAIL_G, EMBED), jnp.float32),
        pltpu.SemaphoreType.DMA,
        pltpu.SemaphoreType.DMA,
        pltpu.SemaphoreType.DMA,
        pltpu.SemaphoreType.DMA,
    ],
)
def _edge_assemble(emb_hbm, src_hbm, dst_hbm, out_hbm,
                   idx0, idx1, rows0, rows1, idxt, rowst,
                   gsem0, gsem1, wsem0, wsem1):
    idx = (idx0, idx1)
    rows = (rows0, rows1)
    gsem = (gsem0, gsem1)
    wsem = (wsem0, wsem1)
    cid = lax.axis_index("c")
    sid = lax.axis_index("s")
    base = (sid * NC + cid) * EPW_G

    def sweep(idx_hbm, col0):
        def out_slice(b, i):
            off = base + i * CHUNK
            return out_hbm.at[pl.ds(off, CHUNK), pl.ds(col0, EMBED)]

        def fire_gather(b, i):
            off = base + i * CHUNK
            pltpu.sync_copy(idx_hbm.at[pl.ds(off, CHUNK)], idx[b])
            pltpu.async_copy(emb_hbm.at[idx[b]], rows[b], gsem[b])

        fire_gather(0, 0)
        fire_gather(1, 1)

        @pl.loop(0, NFULL_G, step=2)
        def _pair(g):
            for b in range(2):
                i = g + b

                @pl.when(i < NFULL_G)
                def _():
                    pltpu.make_async_copy(
                        emb_hbm.at[idx[b]], rows[b], gsem[b]).wait()
                    pltpu.async_copy(rows[b], out_slice(b, i), wsem[b])

                    @pl.when(i + 2 < NFULL_G)
                    def _():
                        pltpu.make_async_copy(
                            rows[b], out_slice(b, i), wsem[b]).wait()
                        fire_gather(b, i + 2)

        for b in range(2):
            pltpu.make_async_copy(rows[b], out_slice(b, 0), wsem[b]).wait()
        offt = base + NFULL_G * CHUNK
        pltpu.sync_copy(idx_hbm.at[pl.ds(offt, TAIL_G)], idxt)
        pltpu.async_copy(emb_hbm.at[idxt], rowst, gsem[0]).wait()
        pltpu.sync_copy(rowst, out_hbm.at[pl.ds(offt, TAIL_G),
                                          pl.ds(col0, EMBED)])

    sweep(src_hbm, 0)
    sweep(dst_hbm, EMBED)


_BLK_N = 512


def _layer0_tc(xp, aggp, WsT, WnT, b, g, beta):
    def body(x_ref, p_ref, ws_ref, wn_ref, b_ref, g_ref, beta_ref,
             h0_ref, h1_ref, h2_ref, h3_ref, rc_ref):
        p = p_ref[0, 0] + p_ref[0, 1]
        cnt = p[:, 12:13]
        rc = 1.0 / jnp.maximum(cnt, 1.0)
        agg = p[:, 0:12] * rc
        h = (jnp.dot(x_ref[...], ws_ref[...], preferred_element_type=jnp.float32)
             + jnp.dot(agg, wn_ref[...], preferred_element_type=jnp.float32)
             + b_ref[...])
        mu = jnp.mean(h, axis=-1, keepdims=True)
        var = jnp.mean((h - mu) ** 2, axis=-1, keepdims=True)
        h = (h - mu) / jnp.sqrt(var + 1e-5) * g_ref[...] + beta_ref[...]
        h = jnp.maximum(h, 0.0)
        h0_ref[...] = h[:, 0:16]
        h1_ref[...] = h[:, 16:32]
        h2_ref[...] = h[:, 32:48]
        h3_ref[...] = h[:, 48:64]
        rc_ref[...] = rc

    grid = (NP // _BLK_N,)
    hspec = pl.BlockSpec((_BLK_N, 16), lambda i: (i, 0))
    hshape = jax.ShapeDtypeStruct((NP, 16), jnp.float32)
    return pl.pallas_call(
        body,
        grid=grid,
        in_specs=[
            pl.BlockSpec((_BLK_N, NODE_DIM), lambda i: (i, 0)),
            pl.BlockSpec((1, NC, _BLK_N, 16), lambda i: (0, 0, i, 0)),
            pl.BlockSpec((NODE_DIM, HIDDEN), lambda i: (0, 0)),
            pl.BlockSpec((NODE_DIM, HIDDEN), lambda i: (0, 0)),
            pl.BlockSpec((HIDDEN,), lambda i: (0,)),
            pl.BlockSpec((HIDDEN,), lambda i: (0,)),
            pl.BlockSpec((HIDDEN,), lambda i: (0,)),
        ],
        out_specs=[hspec, hspec, hspec, hspec,
                   pl.BlockSpec((_BLK_N, 1), lambda i: (i, 0))],
        out_shape=[hshape, hshape, hshape, hshape,
                   jax.ShapeDtypeStruct((NP, 1), jnp.float32)],
    )(xp, aggp, WsT, WnT, b, g, beta)


def _layer1_tc(h0, h1, h2, h3, aggp, rcnt, WsT, WnT, b, g, beta):
    def body(h0_ref, h1_ref, h2_ref, h3_ref, p_ref, rc_ref, ws_ref, wn_ref,
             b_ref, g_ref, beta_ref, out_ref):
        rc = rc_ref[...]
        h = jnp.concatenate(
            [h0_ref[...], h1_ref[...], h2_ref[...], h3_ref[...]], axis=1)
        agg = jnp.concatenate(
            [p_ref[k, 0] + p_ref[k, 1] for k in range(4)], axis=1) * rc
        e = (jnp.dot(h, ws_ref[...], preferred_element_type=jnp.float32)
             + jnp.dot(agg, wn_ref[...], preferred_element_type=jnp.float32)
             + b_ref[...])
        mu = jnp.mean(e, axis=-1, keepdims=True)
        var = jnp.mean((e - mu) ** 2, axis=-1, keepdims=True)
        e = (e - mu) / jnp.sqrt(var + 1e-5) * g_ref[...] + beta_ref[...]
        out_ref[...] = jnp.maximum(e, 0.0)

    grid = (NP // _BLK_N,)
    hspec = pl.BlockSpec((_BLK_N, 16), lambda i: (i, 0))
    return pl.pallas_call(
        body,
        grid=grid,
        in_specs=[
            hspec, hspec, hspec, hspec,
            pl.BlockSpec((4, NC, _BLK_N, 16), lambda i: (0, 0, i, 0)),
            pl.BlockSpec((_BLK_N, 1), lambda i: (i, 0)),
            pl.BlockSpec((HIDDEN, EMBED), lambda i: (0, 0)),
            pl.BlockSpec((HIDDEN, EMBED), lambda i: (0, 0)),
            pl.BlockSpec((EMBED,), lambda i: (0,)),
            pl.BlockSpec((EMBED,), lambda i: (0,)),
            pl.BlockSpec((EMBED,), lambda i: (0,)),
        ],
        out_specs=pl.BlockSpec((_BLK_N, EMBED), lambda i: (i, 0)),
        out_shape=jax.ShapeDtypeStruct((NP, EMBED), jnp.float32),
    )(h0, h1, h2, h3, aggp, rcnt, WsT, WnT, b, g, beta)


_BLK_E = 3200


def _edge_mlp_tc(hsd, edge_attrT, We1, be1, We2, be2, Wd1, bd1, Wd2, bd2):
    def body(hsd_ref, ea_ref, we1_ref, be1_ref, we2_ref, be2_ref,
             wd1_ref, bd1_ref, wd2_ref, bd2_ref, rec_ref, er_ref):
        hsdT = hsd_ref[...].T
        er = jnp.concatenate([hsdT, ea_ref[...]], axis=0)
        er_ref[...] = er
        l1 = jnp.maximum(
            jnp.dot(we1_ref[...], er, preferred_element_type=jnp.float32)
            + be1_ref[...][:, None], 0.0)
        lat = jnp.maximum(
            jnp.dot(we2_ref[...], l1, preferred_element_type=jnp.float32)
            + be2_ref[...][:, None], 0.0)
        d1 = jnp.maximum(
            jnp.dot(wd1_ref[...], lat, preferred_element_type=jnp.float32)
            + bd1_ref[...][:, None], 0.0)
        rec_ref[...] = (jnp.dot(wd2_ref[...], d1, preferred_element_type=jnp.float32)
                        + bd2_ref[...][:, None])

    grid = (N_EDGES // _BLK_E,)
    full = lambda shape: pl.BlockSpec(shape, lambda i: tuple(0 for _ in shape))
    return pl.pallas_call(
        body,
        grid=grid,
        in_specs=[
            pl.BlockSpec((_BLK_E, 2 * EMBED), lambda i: (i, 0)),
            pl.BlockSpec((10, _BLK_E), lambda i: (0, i)),
            full((HIDDEN, EDGE_REPR)),
            full((HIDDEN,)),
            full((EMBED, HIDDEN)),
            full((EMBED,)),
            full((HIDDEN, EMBED)),
            full((HIDDEN,)),
            full((EDGE_REPR, HIDDEN)),
            full((EDGE_REPR,)),
        ],
        out_specs=[
            pl.BlockSpec((EDGE_REPR, _BLK_E), lambda i: (0, i)),
            pl.BlockSpec((EDGE_REPR, _BLK_E), lambda i: (0, i)),
        ],
        out_shape=[
            jax.ShapeDtypeStruct((EDGE_REPR, N_EDGES), jnp.float32),
            jax.ShapeDtypeStruct((EDGE_REPR, N_EDGES), jnp.float32),
        ],
    )(hsd, edge_attrT, We1, be1, We2, be2, Wd1, bd1, Wd2, bd2)


_segsum1 = _make_segsum(1)
_segsum4 = _make_segsum(4)


def kernel(node_feats, edge_attr, W_self0, W_neigh0, b0, g0, beta0,
           W_self1, W_neigh1, b1, g1, beta1, We1, be1, We2, be2,
           Wd1, bd1, Wd2, bd2, edge_index, src_nodes, dst_nodes):
    s = edge_index[0]
    d = edge_index[1]

    npad = E_PAD - N_EDGES
    pad_ids = N_NODES + (jnp.arange(npad, dtype=jnp.int32) % N_DUMMY)
    s_pad = jnp.concatenate([s, pad_ids])
    d_pad = jnp.concatenate([d, pad_ids])

    x16 = jnp.concatenate(
        [node_feats,
         jnp.ones((N_NODES, 1), jnp.float32),
         jnp.zeros((N_NODES, 3), jnp.float32)], axis=1)
    x16 = jnp.pad(x16, ((0, NP - N_NODES), (0, 0)))
    xp = jnp.pad(node_feats, ((0, NP - N_NODES), (0, 0)))

    zeros16 = jnp.zeros((RPT, 16), jnp.float32)

    agg0p = _segsum1(x16, s_pad, d_pad, zeros16)
    h0, h1, h2, h3, rcnt = _layer0_tc(
        xp, agg0p, W_self0.T, W_neigh0.T, b0, g0, beta0)

    agg1p = _segsum4(h0, h1, h2, h3, s_pad, d_pad, zeros16)
    emb = _layer1_tc(h0, h1, h2, h3, agg1p, rcnt,
                     W_self1.T, W_neigh1.T, b1, g1, beta1)

    hsd = _edge_assemble(emb, src_nodes, dst_nodes)
    reconT, edge_repT = _edge_mlp_tc(
        hsd, edge_attr.T, We1, be1, We2, be2, Wd1, bd1, Wd2, bd2)
    return (reconT.T, edge_repT.T)
